# baseline (speedup 1.0000x reference)
import functools

import jax
import jax.numpy as jnp
from jax.experimental import pallas as pl

_B, _N = 4096, 50
_D1, _D2 = 32, 32
_H, _O, _R = 64, 32, 8


def _leaky(x):
    return jnp.where(x > 0, x, 0.01 * x)


def _fused_kernel(u_ref, i_ref, a_ref, o_ref, s_ref,
                  W1c_ref, b1_ref, W2c_ref, b2_ref,
                  Wu1_ref, bu1_ref, Wu2_ref, bu2_ref,
                  E1_ref, E2_ref,
                  out_ref, *, bb, n):
    rows = bb * n
    ao = jnp.concatenate([a_ref[...], o_ref[...]], axis=-1)

    s = s_ref[...]
    rid = jax.lax.broadcasted_iota(jnp.int32, (rows, _R), 1)
    m = (s == rid).astype(jnp.float32)

    z1 = jnp.dot(ao, W1c_ref[...], preferred_element_type=jnp.float32)
    M1 = jnp.dot(m, E1_ref[...], preferred_element_type=jnp.float32)
    z1m = z1 * M1
    h_pre = z1m[:, :_H]
    for r in range(1, _R):
        h_pre = h_pre + z1m[:, r * _H:(r + 1) * _H]
    h_pre = h_pre + jnp.dot(m, b1_ref[...], preferred_element_type=jnp.float32)
    h_sel = _leaky(h_pre)

    z2 = jnp.dot(h_sel, W2c_ref[...], preferred_element_type=jnp.float32)
    M2 = jnp.dot(m, E2_ref[...], preferred_element_type=jnp.float32)
    z2m = z2 * M2
    o_pre = z2m[:, :_O]
    for r in range(1, _R):
        o_pre = o_pre + z2m[:, r * _O:(r + 1) * _O]
    o_pre = o_pre + jnp.dot(m, b2_ref[...], preferred_element_type=jnp.float32)
    o_sel = _leaky(o_pre)

    ui_in = jnp.concatenate([u_ref[...], i_ref[...]], axis=-1)
    hu = _leaky(jnp.dot(ui_in, Wu1_ref[...],
                        preferred_element_type=jnp.float32) + bu1_ref[...])
    ue = _leaky(jnp.dot(hu, Wu2_ref[...],
                        preferred_element_type=jnp.float32) + bu2_ref[...])
    t_row = jax.lax.broadcasted_iota(jnp.int32, (rows, bb), 0)
    b_col = jax.lax.broadcasted_iota(jnp.int32, (rows, bb), 1)
    E3 = (t_row // n == b_col).astype(jnp.float32)
    ue_rows = jnp.dot(E3, ue, preferred_element_type=jnp.float32)

    out_ref[...] = jnp.sum(o_sel * ue_rows, axis=-1, keepdims=True)


@jax.jit
def kernel(u_emb, i_emb, a_emb, o_emb, s,
           W_ui1, b_ui1, W_ui2, b_ui2, W_ao1, b_ao1, W_ao2, b_ao2):
    BB = 64
    grid = (_B // BB,)
    rows = BB * _N
    a2 = a_emb.reshape(_B * _N, _D1)
    o2 = o_emb.reshape(_B * _N, _D1)
    s2 = s.reshape(_B * _N, 1)

    W1c = jnp.transpose(W_ao1, (1, 0, 2)).reshape(2 * _D1, _R * _H)
    W2c = jnp.transpose(W_ao2, (1, 0, 2)).reshape(_H, _R * _O)
    bu1 = b_ui1.reshape(1, _H)
    bu2 = b_ui2.reshape(1, _O)
    E1 = jnp.repeat(jnp.eye(_R, dtype=jnp.float32), _H, axis=1)
    E2 = jnp.repeat(jnp.eye(_R, dtype=jnp.float32), _O, axis=1)

    full = lambda *shape: pl.BlockSpec(shape, lambda i: (0,) * len(shape))
    out = pl.pallas_call(
        functools.partial(_fused_kernel, bb=BB, n=_N),
        grid=grid,
        in_specs=[
            pl.BlockSpec((BB, _D2), lambda i: (i, 0)),
            pl.BlockSpec((BB, _D2), lambda i: (i, 0)),
            pl.BlockSpec((rows, _D1), lambda i: (i, 0)),
            pl.BlockSpec((rows, _D1), lambda i: (i, 0)),
            pl.BlockSpec((rows, 1), lambda i: (i, 0)),
            full(2 * _D1, _R * _H),
            full(_R, _H),
            full(_H, _R * _O),
            full(_R, _O),
            full(2 * _D2, _H),
            full(1, _H),
            full(_H, _O),
            full(1, _O),
            full(_R, _R * _H),
            full(_R, _R * _O),
        ],
        out_specs=pl.BlockSpec((rows, 1), lambda i: (i, 0)),
        out_shape=jax.ShapeDtypeStruct((_B * _N, 1), jnp.float32),
    )(u_emb, i_emb, a2, o2, s2, W1c, b_ao1, W2c, b_ao2,
      W_ui1, bu1, W_ui2, bu2, E1, E2)
    return out.reshape(_B, _N)


# speedup vs baseline: 1.4025x; 1.4025x over previous
import functools

import jax
import jax.numpy as jnp
from jax.experimental import pallas as pl

_B, _N = 4096, 50
_D1, _D2 = 32, 32
_H, _O, _R = 64, 32, 8


def _leaky(x):
    return jnp.where(x > 0, x, 0.01 * x)


def _fused_kernel(u_ref, i_ref, a_ref, o_ref, s_ref,
                  W1c_ref, b1_ref, W2c_ref, b2_ref,
                  Wu1_ref, bu1_ref, Wu2_ref, bu2_ref,
                  E1_ref, E2_ref,
                  out_ref, *, bb, n):
    rows = bb * n
    a = a_ref[...].reshape(rows, _D1)
    o = o_ref[...].reshape(rows, _D1)
    ao = jnp.concatenate([a, o], axis=-1)

    rid = jax.lax.broadcasted_iota(jnp.int32, (bb, n, _R), 2)
    m = (s_ref[...][:, :, None] == rid).astype(jnp.float32).reshape(rows, _R)

    z1 = jnp.dot(ao, W1c_ref[...], preferred_element_type=jnp.float32)
    M1 = jnp.dot(m, E1_ref[...], preferred_element_type=jnp.float32)
    z1m = z1 * M1
    h_pre = z1m[:, :_H]
    for r in range(1, _R):
        h_pre = h_pre + z1m[:, r * _H:(r + 1) * _H]
    h_pre = h_pre + jnp.dot(m, b1_ref[...], preferred_element_type=jnp.float32)
    h_sel = _leaky(h_pre)

    z2 = jnp.dot(h_sel, W2c_ref[...], preferred_element_type=jnp.float32)
    M2 = jnp.dot(m, E2_ref[...], preferred_element_type=jnp.float32)
    z2m = z2 * M2
    o_pre = z2m[:, :_O]
    for r in range(1, _R):
        o_pre = o_pre + z2m[:, r * _O:(r + 1) * _O]
    o_pre = o_pre + jnp.dot(m, b2_ref[...], preferred_element_type=jnp.float32)
    o_sel = _leaky(o_pre)

    ui_in = jnp.concatenate([u_ref[...], i_ref[...]], axis=-1)
    hu = _leaky(jnp.dot(ui_in, Wu1_ref[...],
                        preferred_element_type=jnp.float32) + bu1_ref[...])
    ue = _leaky(jnp.dot(hu, Wu2_ref[...],
                        preferred_element_type=jnp.float32) + bu2_ref[...])

    o3 = o_sel.reshape(bb, n, _O)
    out_ref[...] = jnp.sum(o3 * ue[:, None, :], axis=-1)


@jax.jit
def kernel(u_emb, i_emb, a_emb, o_emb, s,
           W_ui1, b_ui1, W_ui2, b_ui2, W_ao1, b_ao1, W_ao2, b_ao2):
    BB = 64
    grid = (_B // BB,)

    W1c = jnp.transpose(W_ao1, (1, 0, 2)).reshape(2 * _D1, _R * _H)
    W2c = jnp.transpose(W_ao2, (1, 0, 2)).reshape(_H, _R * _O)
    bu1 = b_ui1.reshape(1, _H)
    bu2 = b_ui2.reshape(1, _O)
    E1 = jnp.repeat(jnp.eye(_R, dtype=jnp.float32), _H, axis=1)
    E2 = jnp.repeat(jnp.eye(_R, dtype=jnp.float32), _O, axis=1)

    full = lambda *shape: pl.BlockSpec(shape, lambda i: (0,) * len(shape))
    out = pl.pallas_call(
        functools.partial(_fused_kernel, bb=BB, n=_N),
        grid=grid,
        in_specs=[
            pl.BlockSpec((BB, _D2), lambda i: (i, 0)),
            pl.BlockSpec((BB, _D2), lambda i: (i, 0)),
            pl.BlockSpec((BB, _N, _D1), lambda i: (i, 0, 0)),
            pl.BlockSpec((BB, _N, _D1), lambda i: (i, 0, 0)),
            pl.BlockSpec((BB, _N), lambda i: (i, 0)),
            full(2 * _D1, _R * _H),
            full(_R, _H),
            full(_H, _R * _O),
            full(_R, _O),
            full(2 * _D2, _H),
            full(1, _H),
            full(_H, _O),
            full(1, _O),
            full(_R, _R * _H),
            full(_R, _R * _O),
        ],
        out_specs=pl.BlockSpec((BB, _N), lambda i: (i, 0)),
        out_shape=jax.ShapeDtypeStruct((_B, _N), jnp.float32),
    )(u_emb, i_emb, a_emb, o_emb, s, W1c, b_ao1, W2c, b_ao2,
      W_ui1, bu1, W_ui2, bu2, E1, E2)
    return out
